# baseline (speedup 1.0000x reference)
import functools
import math

import jax
import jax.numpy as jnp
from jax import lax
from jax.experimental import pallas as pl
from jax.experimental.pallas import tpu as pltpu
from jax.experimental.pallas import tpu_sc as plsc

D_MODEL = 64
SCALE = math.sqrt(D_MODEL)

_info = plsc.get_sparse_core_info()
_NC, _NS = _info.num_cores, _info.num_subcores
_NW = _NC * _NS

_CHUNK = 128


@functools.partial(jax.jit, static_argnames=("n_rows",))
def _embed_lookup(idx2d, table, n_rows):
    n_per_w = n_rows // _NW
    n_chunks = n_per_w // _CHUNK
    idx_rows_per_w = n_per_w // 128

    mesh = plsc.VectorSubcoreMesh(core_axis_name="c", subcore_axis_name="s")

    @functools.partial(
        pl.kernel,
        mesh=mesh,
        out_type=jax.ShapeDtypeStruct((n_rows, D_MODEL), jnp.float32),
        scratch_types=[
            pltpu.VMEM((idx_rows_per_w, 128), jnp.int32),
            pltpu.VMEM((_CHUNK, D_MODEL), jnp.float32),
            pltpu.SemaphoreType.DMA,
        ],
        compiler_params=pltpu.CompilerParams(use_tc_tiling_on_sc=False),
    )
    def k(idx_hbm, table_hbm, out_hbm, idx_v, rows_v, sem):
        wid = lax.axis_index("s") * _NC + lax.axis_index("c")
        pltpu.sync_copy(idx_hbm.at[pl.ds(wid * idx_rows_per_w, idx_rows_per_w)],
                        idx_v)
        row_base = wid * n_per_w

        def chunk_body(g, carry):
            pltpu.async_copy(table_hbm.at[idx_v.at[g]], rows_v, sem).wait()
            def scale_body(r, c2):
                for j in range(D_MODEL // 16):
                    sl = pl.ds(j * 16, 16)
                    rows_v[r, sl] = rows_v[r, sl] * SCALE
                return c2
            lax.fori_loop(0, _CHUNK, scale_body, 0)
            pltpu.sync_copy(rows_v,
                            out_hbm.at[pl.ds(row_base + g * _CHUNK, _CHUNK)])
            return carry

        lax.fori_loop(0, n_chunks, chunk_body, 0)

    return k(idx2d, table)


def kernel(x, table):
    b, s = x.shape
    n_rows = b * s
    idx2d = x.reshape(n_rows // 128, 128).astype(jnp.int32)
    out = _embed_lookup(idx2d, table, n_rows)
    return out.reshape(b, s, D_MODEL)


# speedup vs baseline: 1.2079x; 1.2079x over previous
import functools
import math

import jax
import jax.numpy as jnp
from jax import lax
from jax.experimental import pallas as pl
from jax.experimental.pallas import tpu as pltpu
from jax.experimental.pallas import tpu_sc as plsc

D_MODEL = 64
SCALE = math.sqrt(D_MODEL)

_info = plsc.get_sparse_core_info()
_NC, _NS = _info.num_cores, _info.num_subcores
_NW = _NC * _NS

_IDXW = 128
_GPC = 2
_CHUNK = _IDXW * _GPC
_NBUF = 4


@functools.partial(jax.jit, static_argnames=("n_rows",))
def _embed_lookup(idx2d, table, n_rows):
    n_per_w = n_rows // _NW
    n_chunks = n_per_w // _CHUNK
    idx_rows_per_w = n_per_w // _IDXW
    assert n_chunks % _NBUF == 0 and n_chunks >= 2 * _NBUF

    mesh = plsc.VectorSubcoreMesh(core_axis_name="c", subcore_axis_name="s")

    @functools.partial(
        pl.kernel,
        mesh=mesh,
        out_type=jax.ShapeDtypeStruct((n_rows, D_MODEL), jnp.float32),
        scratch_types=[
            pltpu.VMEM((idx_rows_per_w, _IDXW), jnp.int32),
            [pltpu.VMEM((_CHUNK, D_MODEL), jnp.float32)] * _NBUF,
            [pltpu.SemaphoreType.DMA] * _NBUF,
            [pltpu.SemaphoreType.DMA] * _NBUF,
        ],
        compiler_params=pltpu.CompilerParams(use_tc_tiling_on_sc=False),
    )
    def k(idx_hbm, table_hbm, out_hbm, idx_v, bufs, gsems, osems):
        wid = lax.axis_index("s") * _NC + lax.axis_index("c")
        pltpu.sync_copy(idx_hbm.at[pl.ds(wid * idx_rows_per_w, idx_rows_per_w)],
                        idx_v)
        row_base = wid * n_per_w

        def issue_gather(i, b):
            for j in range(_GPC):
                pltpu.async_copy(table_hbm.at[idx_v.at[_GPC * i + j]],
                                 bufs[b].at[pl.ds(j * _IDXW, _IDXW)],
                                 gsems[b])

        def wait_gather(b):
            for j in range(_GPC):
                pltpu.make_async_copy(table_hbm.at[idx_v.at[0]],
                                      bufs[b].at[pl.ds(j * _IDXW, _IDXW)],
                                      gsems[b]).wait()

        def issue_out(i, b):
            pltpu.async_copy(bufs[b],
                             out_hbm.at[pl.ds(row_base + i * _CHUNK, _CHUNK)],
                             osems[b])

        def wait_out(b):
            pltpu.make_async_copy(bufs[b],
                                  out_hbm.at[pl.ds(row_base, _CHUNK)],
                                  osems[b]).wait()

        def scale(b):
            buf = bufs[b]
            def scale_body(r8, c2):
                for r in range(8):
                    for j in range(D_MODEL // 16):
                        sl = (r8 * 8 + r, pl.ds(j * 16, 16))
                        buf[sl] = buf[sl] * SCALE
                return c2
            lax.fori_loop(0, _CHUNK // 8, scale_body, 0)

        def step_b(i, b):
            wait_gather(b)
            scale(b)
            issue_out(i, b)

        issue_gather(0, 0)
        issue_gather(1, 1)
        issue_gather(2, 2)
        step_b(0, 0)
        issue_gather(3, 3)
        step_b(1, 1)
        wait_out(0)
        issue_gather(4, 0)
        step_b(2, 2)
        wait_out(1)
        issue_gather(5, 1)
        step_b(3, 3)

        def loop_body(kk, carry):
            i0 = 4 * kk
            for m in range(4):
                i = i0 + m
                bp = (m + 2) % 4
                wait_out(bp)
                issue_gather(i + 2, bp)
                step_b(i, m)
            return carry

        lax.fori_loop(1, n_chunks // 4 - 1, loop_body, 0)

        nl = n_chunks - 4
        wait_out(2)
        issue_gather(nl + 2, 2)
        step_b(nl + 0, 0)
        wait_out(3)
        issue_gather(nl + 3, 3)
        step_b(nl + 1, 1)
        step_b(nl + 2, 2)
        step_b(nl + 3, 3)
        for b in range(_NBUF):
            wait_out(b)

    return k(idx2d, table)


def kernel(x, table):
    b, s = x.shape
    n_rows = b * s
    idx2d = x.reshape(n_rows // _IDXW, _IDXW).astype(jnp.int32)
    out = _embed_lookup(idx2d, table, n_rows)
    return out.reshape(b, s, D_MODEL)
